# Initial kernel scaffold; baseline (speedup 1.0000x reference)
import jax
import jax.numpy as jnp
from jax.experimental import pallas as pl


def kernel(s, in_w, in_b, W, a_src, a_dst, Wo, ln_g, ln_b, fin_g, fin_b):
    raise NotImplementedError("write your pallas kernel here")


# speedup vs baseline: 55.3015x; 55.3015x over previous
import jax
import jax.numpy as jnp
from jax.experimental import pallas as pl
from jax.experimental.pallas import tpu as pltpu

_B = 16
_T = 2048
_IN = 3
_D = 256
_H = 8
_DH = _D // _H
_DEPTH = 3


def _ln(x, g, b):
    mu = jnp.mean(x, axis=-1, keepdims=True)
    xc = x - mu
    v = jnp.mean(xc * xc, axis=-1, keepdims=True)
    return xc * jax.lax.rsqrt(v + 1e-5) * g + b


def _lrelu(x):
    return jnp.where(x >= 0, x, 0.2 * x)


def _tgnn_kernel(s_ref, in_w_ref, in_b_ref, W_ref, A_src_ref, A_dst_ref,
                 R_ref, Wo_ref, ln_g_ref, ln_b_ref, fin_ref, out_ref):
    s = s_ref[0]
    h = (s[:, 0:1] * in_w_ref[0:1, :]
         + s[:, 1:2] * in_w_ref[1:2, :]
         + s[:, 2:3] * in_w_ref[2:3, :]
         + in_b_ref[0:1, :])

    row = jax.lax.broadcasted_iota(jnp.int32, (_T, 1), 0)
    has_prev = row >= 1
    has_next = row <= _T - 2
    R = R_ref[...]

    for l in range(_DEPTH):
        hw = jnp.dot(h, W_ref[l], preferred_element_type=jnp.float32)
        es = jnp.dot(hw, A_src_ref[l], preferred_element_type=jnp.float32)
        ed = jnp.dot(hw, A_dst_ref[l], preferred_element_type=jnp.float32)

        es_up = jnp.concatenate([es[:1], es[:-1]], axis=0)
        es_dn = jnp.concatenate([es[1:], es[-1:]], axis=0)
        e_self = _lrelu(es + ed)
        e_prev = jnp.where(has_prev, _lrelu(es_up + ed), -1e30)
        e_next = jnp.where(has_next, _lrelu(es_dn + ed), -1e30)

        m = jnp.maximum(e_self, jnp.maximum(e_prev, e_next))
        x_self = jnp.exp(e_self - m)
        x_prev = jnp.where(has_prev, jnp.exp(e_prev - m), 0.0)
        x_next = jnp.where(has_next, jnp.exp(e_next - m), 0.0)
        inv = 1.0 / (x_self + x_prev + x_next + 1e-9)

        af_self = jnp.dot(x_self * inv, R, preferred_element_type=jnp.float32)
        af_prev = jnp.dot(x_prev * inv, R, preferred_element_type=jnp.float32)
        af_next = jnp.dot(x_next * inv, R, preferred_element_type=jnp.float32)

        hw_up = jnp.concatenate([hw[:1], hw[:-1]], axis=0)
        hw_dn = jnp.concatenate([hw[1:], hw[-1:]], axis=0)
        agg = af_self * hw + af_prev * hw_up + af_next * hw_dn

        act = jnp.where(agg > 0, agg, jnp.exp(jnp.minimum(agg, 0.0)) - 1.0)
        out = jnp.dot(act, Wo_ref[l], preferred_element_type=jnp.float32)
        h = _ln(h + out, ln_g_ref[l:l + 1, :], ln_b_ref[l:l + 1, :])

    out_ref[0] = _ln(h, fin_ref[0:1, :], fin_ref[1:2, :])


def kernel(s, in_w, in_b, W, a_src, a_dst, Wo, ln_g, ln_b, fin_g, fin_b):
    eye = jnp.eye(_H, dtype=jnp.float32)
    A_src = (a_src[:, :, :, None] * eye[None, :, None, :]).reshape(_DEPTH, _D, _H)
    A_dst = (a_dst[:, :, :, None] * eye[None, :, None, :]).reshape(_DEPTH, _D, _H)
    R = jnp.repeat(eye, _DH, axis=1)
    fin = jnp.stack([fin_g, fin_b])

    return pl.pallas_call(
        _tgnn_kernel,
        grid=(_B,),
        in_specs=[
            pl.BlockSpec((1, _T, _IN), lambda b: (b, 0, 0)),
            pl.BlockSpec((_IN, _D), lambda b: (0, 0)),
            pl.BlockSpec((1, _D), lambda b: (0, 0)),
            pl.BlockSpec((_DEPTH, _D, _D), lambda b: (0, 0, 0)),
            pl.BlockSpec((_DEPTH, _D, _H), lambda b: (0, 0, 0)),
            pl.BlockSpec((_DEPTH, _D, _H), lambda b: (0, 0, 0)),
            pl.BlockSpec((_H, _D), lambda b: (0, 0)),
            pl.BlockSpec((_DEPTH, _D, _D), lambda b: (0, 0, 0)),
            pl.BlockSpec((_DEPTH, _D), lambda b: (0, 0)),
            pl.BlockSpec((_DEPTH, _D), lambda b: (0, 0)),
            pl.BlockSpec((2, _D), lambda b: (0, 0)),
        ],
        out_specs=pl.BlockSpec((1, _T, _D), lambda b: (b, 0, 0)),
        out_shape=jax.ShapeDtypeStruct((_B, _T, _D), jnp.float32),
        compiler_params=pltpu.CompilerParams(
            dimension_semantics=("arbitrary",)),
    )(s, in_w, in_b[None, :], W, A_src, A_dst, R, Wo, ln_g, ln_b, fin)
